# Initial kernel scaffold; baseline (speedup 1.0000x reference)
import jax
import jax.numpy as jnp
from jax.experimental import pallas as pl


def kernel(users, items, mlp_user_table, mlp_item_table, gmf_user_table, gmf_item_table, mlp_W0, mlp_b0, mlp_W1, mlp_b1, mlp_W2, mlp_b2, mlp_fc_w, mlp_fc_b, gmf_fc_w, gmf_fc_b):
    raise NotImplementedError("write your pallas kernel here")


# speedup vs baseline: 4.0872x; 4.0872x over previous
import functools

import jax
import jax.numpy as jnp
from jax import lax
from jax.experimental import pallas as pl
from jax.experimental.pallas import tpu as pltpu
from jax.experimental.pallas import tpu_sc as plsc

B = 16384
EMB = 32
NC = 2
NS = 16
NW = NC * NS
BPW = B // NW

_HI = jax.lax.Precision.HIGHEST


def _sc_gather(mlp_u, mlp_i, gmf_u, gmf_i, users, items):
    mesh = plsc.VectorSubcoreMesh(core_axis_name="c", subcore_axis_name="s")
    row = jax.ShapeDtypeStruct((B, EMB), jnp.float32)

    @functools.partial(
        pl.kernel,
        mesh=mesh,
        out_type=[row, row, row, row],
        scratch_types=[
            pltpu.VMEM((BPW,), jnp.int32),
            pltpu.VMEM((BPW,), jnp.int32),
            pltpu.VMEM((BPW, EMB), jnp.float32),
            pltpu.VMEM((BPW, EMB), jnp.float32),
            pltpu.VMEM((BPW, EMB), jnp.float32),
            pltpu.VMEM((BPW, EMB), jnp.float32),
            pltpu.SemaphoreType.DMA,
        ],
    )
    def k(mu_hbm, mi_hbm, gu_hbm, gi_hbm, u_hbm, it_hbm,
          omu, omi, ogu, ogi,
          uidx, iidx, mu_v, mi_v, gu_v, gi_v, sem):
        wid = lax.axis_index("s") * NC + lax.axis_index("c")
        base = wid * BPW
        pltpu.sync_copy(u_hbm.at[pl.ds(base, BPW)], uidx)
        pltpu.sync_copy(it_hbm.at[pl.ds(base, BPW)], iidx)
        c1 = pltpu.async_copy(mu_hbm.at[uidx], mu_v, sem)
        c2 = pltpu.async_copy(mi_hbm.at[iidx], mi_v, sem)
        c3 = pltpu.async_copy(gu_hbm.at[uidx], gu_v, sem)
        c4 = pltpu.async_copy(gi_hbm.at[iidx], gi_v, sem)
        c1.wait()
        c2.wait()
        c3.wait()
        c4.wait()
        pltpu.sync_copy(mu_v, omu.at[pl.ds(base, BPW)])
        pltpu.sync_copy(mi_v, omi.at[pl.ds(base, BPW)])
        pltpu.sync_copy(gu_v, ogu.at[pl.ds(base, BPW)])
        pltpu.sync_copy(gi_v, ogi.at[pl.ds(base, BPW)])

    return k(mlp_u, mlp_i, gmf_u, gmf_i, users, items)


def _tc_body(mu, mi, gu, gi, w0a, w0b, b0, w1, b1, w2, b2, wm, wg, bias, out):
    h = jnp.dot(mu[...], w0a[...], precision=_HI) + jnp.dot(mi[...], w0b[...], precision=_HI)
    h = jnp.maximum(h + b0[...], 0.0)
    h = jnp.maximum(jnp.dot(h, w1[...], precision=_HI) + b1[...], 0.0)
    h = jnp.maximum(jnp.dot(h, w2[...], precision=_HI) + b2[...], 0.0)
    y = jnp.sum(h * wm[...], axis=1)
    y = y + jnp.sum((gu[...] * gi[...]) * wg[...], axis=1)
    out[...] = (y + bias[0, 0])[:, None]


def _tc_dense(mu, mi, gu, gi, w0a, w0b, b0, w1, b1, w2, b2, wm, wg, bias):
    blk = 2048
    grid = (B // blk,)
    emb_spec = pl.BlockSpec((blk, EMB), lambda i: (i, 0))

    def full(shape):
        return pl.BlockSpec(shape, lambda i: (0,) * len(shape))

    return pl.pallas_call(
        _tc_body,
        grid=grid,
        in_specs=[
            emb_spec, emb_spec, emb_spec, emb_spec,
            full((EMB, 128)), full((EMB, 128)), full((1, 128)),
            full((128, 64)), full((1, 64)),
            full((64, 32)), full((1, 32)),
            full((1, 32)), full((1, 32)), full((1, 1)),
        ],
        out_specs=pl.BlockSpec((blk, 1), lambda i: (i, 0)),
        out_shape=jax.ShapeDtypeStruct((B, 1), jnp.float32),
    )(mu, mi, gu, gi, w0a, w0b, b0, w1, b1, w2, b2, wm, wg, bias)


def kernel(users, items, mlp_user_table, mlp_item_table, gmf_user_table,
           gmf_item_table, mlp_W0, mlp_b0, mlp_W1, mlp_b1, mlp_W2, mlp_b2,
           mlp_fc_w, mlp_fc_b, gmf_fc_w, gmf_fc_b):
    users = users.astype(jnp.int32)
    items = items.astype(jnp.int32)
    mu = jnp.take(mlp_user_table, users, axis=0)
    mi = jnp.take(mlp_item_table, items, axis=0)
    gu = jnp.take(gmf_user_table, users, axis=0)
    gi = jnp.take(gmf_item_table, items, axis=0)
    w0a = mlp_W0[:EMB]
    w0b = mlp_W0[EMB:]
    bias = (mlp_fc_b + gmf_fc_b).reshape(1, 1)
    y = _tc_dense(mu, mi, gu, gi,
                  w0a, w0b, mlp_b0.reshape(1, -1),
                  mlp_W1, mlp_b1.reshape(1, -1),
                  mlp_W2, mlp_b2.reshape(1, -1),
                  mlp_fc_w.reshape(1, -1), gmf_fc_w.reshape(1, -1), bias)
    return y[:, 0]
